# baseline (speedup 1.0000x reference)
import functools

import jax
import jax.numpy as jnp
from jax import lax
from jax.experimental import pallas as pl
from jax.experimental.pallas import tpu as pltpu
from jax.experimental.pallas import tpu_sc as plsc

_ALPHA = 0.25
_LOSS_WEIGHT = 1.0

_NC = 2
_NS = 16
_NW = _NC * _NS
_LANES = 16
_UNROLL = 4

_SC_FRAC_NUM = 1
_SC_FRAC_DEN = 4
_TC_COLS = 1024
_TC_BLOCK_ROWS = 128


def _focal_vec(p, tgt):
    t = tgt == 0
    nonneg = p >= 0
    ap = jnp.abs(p)
    e = jnp.exp(-ap)
    r = 1.0 / (1.0 + e)
    er = e * r
    z = e / (e + 2.0)
    w = z * z
    poly = 1.0 + w * (1.0 / 3.0 + w * (1.0 / 5.0 + w * (1.0 / 7.0 + w * (1.0 / 9.0))))
    l1p = 2.0 * z * poly
    q = jnp.where(t, -p, p)
    bce = jnp.maximum(q, 0.0) + l1p
    s = jnp.where(nonneg, r, er)
    pt = jnp.where(t, 1.0 - s, s)
    af = jnp.where(t, _ALPHA, 1.0 - _ALPHA)
    return bce * af * pt * pt


def _sc_partial_sums(predf, target):
    n = predf.shape[0]
    per_w = n // _NW
    vecs = per_w // (_UNROLL * _LANES)
    mesh = plsc.VectorSubcoreMesh(core_axis_name="c", subcore_axis_name="s")

    @functools.partial(
        pl.kernel,
        mesh=mesh,
        out_type=jax.ShapeDtypeStruct((_NW, _LANES), jnp.float32),
        scratch_types=[
            pltpu.VMEM((per_w,), jnp.float32),
            pltpu.VMEM((per_w,), jnp.int32),
            pltpu.VMEM((_LANES,), jnp.float32),
            pltpu.SemaphoreType.DMA,
            pltpu.SemaphoreType.DMA,
        ],
    )
    def sc_loss(pred_hbm, tgt_hbm, out_hbm, pred_v, tgt_v, acc_v, sem_p, sem_t):
        wid = lax.axis_index("s") * _NC + lax.axis_index("c")
        base = wid * per_w
        cp = pltpu.async_copy(pred_hbm.at[pl.ds(base, per_w)], pred_v, sem_p)
        ct = pltpu.async_copy(tgt_hbm.at[pl.ds(base, per_w)], tgt_v, sem_t)
        cp.wait()
        ct.wait()

        zero = jnp.zeros((_LANES,), jnp.float32)

        def body(i, accs):
            b = i * (_UNROLL * _LANES)
            out = []
            for k in range(_UNROLL):
                p = pred_v[pl.ds(b + k * _LANES, _LANES)]
                tg = tgt_v[pl.ds(b + k * _LANES, _LANES)]
                out.append(accs[k] + _focal_vec(p, tg))
            return tuple(out)

        accs = lax.fori_loop(0, vecs, body, (zero,) * _UNROLL)
        acc_v[...] = (accs[0] + accs[1]) + (accs[2] + accs[3])
        pltpu.sync_copy(acc_v, out_hbm.at[wid])

    return sc_loss(predf, target)


def _tc_block(pred_ref, tgt_ref, out_ref):
    p = pred_ref[...]
    t = (tgt_ref[...] == 0).astype(jnp.float32)
    s = jax.nn.sigmoid(p)
    pt = (1.0 - s) * t + s * (1.0 - t)
    fw = (_ALPHA * t + (1.0 - _ALPHA) * (1.0 - t)) * pt * pt
    bce = jnp.maximum(p, 0.0) - p * t + jnp.log1p(jnp.exp(-jnp.abs(p)))
    part = jnp.sum(bce * fw)

    @pl.when(pl.program_id(0) == 0)
    def _():
        out_ref[0, 0] = 0.0

    out_ref[0, 0] += part


def _tc_partial_sum(pred2d, tgt2d):
    rows = pred2d.shape[0]
    grid = rows // _TC_BLOCK_ROWS
    return pl.pallas_call(
        _tc_block,
        grid=(grid,),
        in_specs=[
            pl.BlockSpec((_TC_BLOCK_ROWS, _TC_COLS), lambda i: (i, 0)),
            pl.BlockSpec((_TC_BLOCK_ROWS, _TC_COLS), lambda i: (i, 0)),
        ],
        out_specs=pl.BlockSpec(memory_space=pltpu.SMEM),
        out_shape=jax.ShapeDtypeStruct((1, 1), jnp.float32),
        compiler_params=pltpu.CompilerParams(
            dimension_semantics=("arbitrary",),
        ),
    )(pred2d, tgt2d)


def kernel(pred, target):
    n = pred.shape[0]
    predf = pred.reshape(n)
    sc_n = (n * _SC_FRAC_NUM // _SC_FRAC_DEN) // (_NW * _UNROLL * _LANES) \
        * (_NW * _UNROLL * _LANES)
    tc_n = n - sc_n

    sc_partials = _sc_partial_sums(predf[:sc_n], target[:sc_n])
    tc_partial = _tc_partial_sum(
        predf[sc_n:].reshape(tc_n // _TC_COLS, _TC_COLS),
        target[sc_n:].reshape(tc_n // _TC_COLS, _TC_COLS),
    )
    total = jnp.sum(sc_partials) + tc_partial[0, 0]
    return _LOSS_WEIGHT * (total / n)


# speedup vs baseline: 2.8470x; 2.8470x over previous
import functools

import jax
import jax.numpy as jnp
from jax import lax
from jax.experimental import pallas as pl
from jax.experimental.pallas import tpu as pltpu
from jax.experimental.pallas import tpu_sc as plsc

_ALPHA = 0.25
_LOSS_WEIGHT = 1.0

_NC = 2
_NS = 16
_NW = _NC * _NS
_LANES = 16
_UNROLL = 4

_SC_FRAC_NUM = 1
_SC_FRAC_DEN = 4
_TC_COLS = 1024
_TC_BLOCK_ROWS = 128


def _focal_vec(p, tgt):
    t = tgt == 0
    nonneg = p >= 0
    ap = jnp.abs(p)
    e = jnp.exp(-ap)
    r = 1.0 / (1.0 + e)
    er = e * r
    z = e / (e + 2.0)
    w = z * z
    poly = 1.0 + w * (1.0 / 3.0 + w * (1.0 / 5.0 + w * (1.0 / 7.0 + w * (1.0 / 9.0))))
    l1p = 2.0 * z * poly
    q = jnp.where(t, -p, p)
    bce = jnp.maximum(q, 0.0) + l1p
    s = jnp.where(nonneg, r, er)
    pt = jnp.where(t, 1.0 - s, s)
    af = jnp.where(t, _ALPHA, 1.0 - _ALPHA)
    return bce * af * pt * pt


def _sc_partial_sums(predf, target, sc_n):
    per_w = sc_n // _NW
    vecs = per_w // (_UNROLL * _LANES)
    mesh = plsc.VectorSubcoreMesh(core_axis_name="c", subcore_axis_name="s")

    @functools.partial(
        pl.kernel,
        mesh=mesh,
        out_type=jax.ShapeDtypeStruct((_NW, _LANES), jnp.float32),
        scratch_types=[
            pltpu.VMEM((per_w,), jnp.float32),
            pltpu.VMEM((per_w,), jnp.int32),
            pltpu.VMEM((_LANES,), jnp.float32),
            pltpu.SemaphoreType.DMA,
            pltpu.SemaphoreType.DMA,
        ],
    )
    def sc_loss(pred_hbm, tgt_hbm, out_hbm, pred_v, tgt_v, acc_v, sem_p, sem_t):
        wid = lax.axis_index("s") * _NC + lax.axis_index("c")
        base = wid * per_w
        cp = pltpu.async_copy(pred_hbm.at[pl.ds(base, per_w)], pred_v, sem_p)
        ct = pltpu.async_copy(tgt_hbm.at[pl.ds(base, per_w)], tgt_v, sem_t)
        cp.wait()
        ct.wait()

        zero = jnp.zeros((_LANES,), jnp.float32)

        def body(i, accs):
            b = i * (_UNROLL * _LANES)
            out = []
            for k in range(_UNROLL):
                p = pred_v[pl.ds(b + k * _LANES, _LANES)]
                tg = tgt_v[pl.ds(b + k * _LANES, _LANES)]
                out.append(accs[k] + _focal_vec(p, tg))
            return tuple(out)

        accs = lax.fori_loop(0, vecs, body, (zero,) * _UNROLL)
        acc_v[...] = (accs[0] + accs[1]) + (accs[2] + accs[3])
        pltpu.sync_copy(acc_v, out_hbm.at[wid])

    return sc_loss(predf, target)


def _tc_block(pred_ref, tgt_ref, out_ref):
    p = pred_ref[...]
    t = (tgt_ref[...] == 0).astype(jnp.float32)
    s = jax.nn.sigmoid(p)
    pt = (1.0 - s) * t + s * (1.0 - t)
    fw = (_ALPHA * t + (1.0 - _ALPHA) * (1.0 - t)) * pt * pt
    bce = jnp.maximum(p, 0.0) - p * t + jnp.log1p(jnp.exp(-jnp.abs(p)))
    part = jnp.sum(bce * fw)

    @pl.when(pl.program_id(0) == 0)
    def _():
        out_ref[0, 0] = 0.0

    out_ref[0, 0] += part


def _tc_partial_sum(predf, target, sc_n):
    n = predf.shape[0]
    blk = _TC_BLOCK_ROWS * _TC_COLS
    grid = (n - sc_n) // blk
    first = sc_n // blk
    return pl.pallas_call(
        _tc_block,
        grid=(grid,),
        in_specs=[
            pl.BlockSpec((blk,), lambda i: (first + i,)),
            pl.BlockSpec((blk,), lambda i: (first + i,)),
        ],
        out_specs=pl.BlockSpec(memory_space=pltpu.SMEM),
        out_shape=jax.ShapeDtypeStruct((1, 1), jnp.float32),
        compiler_params=pltpu.CompilerParams(
            dimension_semantics=("arbitrary",),
        ),
    )(predf, target)


def kernel(pred, target):
    n = pred.shape[0]
    predf = pred.reshape(n)
    blk = _TC_BLOCK_ROWS * _TC_COLS
    sc_n = (n * _SC_FRAC_NUM // _SC_FRAC_DEN) // blk * blk

    sc_partials = _sc_partial_sums(predf, target, sc_n)
    tc_partial = _tc_partial_sum(predf, target, sc_n)
    total = jnp.sum(sc_partials) + tc_partial[0, 0]
    return _LOSS_WEIGHT * (total / n)


# speedup vs baseline: 2.8773x; 1.0107x over previous
import functools

import jax
import jax.numpy as jnp
from jax import lax
from jax.experimental import pallas as pl
from jax.experimental.pallas import tpu as pltpu
from jax.experimental.pallas import tpu_sc as plsc

_ALPHA = 0.25
_LOSS_WEIGHT = 1.0

_NC = 2
_NS = 16
_NW = _NC * _NS
_LANES = 16
_UNROLL = 4

_SC_FRAC_NUM = 1
_SC_FRAC_DEN = 4
_TC_COLS = 1024
_TC_BLOCK_ROWS = 256


def _focal_vec(p, tgt):
    t = tgt == 0
    nonneg = p >= 0
    ap = jnp.abs(p)
    e = jnp.exp(-ap)
    r = 1.0 / (1.0 + e)
    er = e * r
    z = e / (e + 2.0)
    w = z * z
    poly = 1.0 + w * (1.0 / 3.0 + w * (1.0 / 5.0 + w * (1.0 / 7.0 + w * (1.0 / 9.0))))
    l1p = 2.0 * z * poly
    q = jnp.where(t, -p, p)
    bce = jnp.maximum(q, 0.0) + l1p
    s = jnp.where(nonneg, r, er)
    pt = jnp.where(t, 1.0 - s, s)
    af = jnp.where(t, _ALPHA, 1.0 - _ALPHA)
    return bce * af * pt * pt


def _sc_partial_sums(predf, target, sc_n):
    per_w = sc_n // _NW
    vecs = per_w // (_UNROLL * _LANES)
    mesh = plsc.VectorSubcoreMesh(core_axis_name="c", subcore_axis_name="s")

    @functools.partial(
        pl.kernel,
        mesh=mesh,
        out_type=jax.ShapeDtypeStruct((_NW, _LANES), jnp.float32),
        scratch_types=[
            pltpu.VMEM((per_w,), jnp.float32),
            pltpu.VMEM((per_w,), jnp.int32),
            pltpu.VMEM((_LANES,), jnp.float32),
            pltpu.SemaphoreType.DMA,
            pltpu.SemaphoreType.DMA,
        ],
    )
    def sc_loss(pred_hbm, tgt_hbm, out_hbm, pred_v, tgt_v, acc_v, sem_p, sem_t):
        wid = lax.axis_index("s") * _NC + lax.axis_index("c")
        base = wid * per_w
        cp = pltpu.async_copy(pred_hbm.at[pl.ds(base, per_w)], pred_v, sem_p)
        ct = pltpu.async_copy(tgt_hbm.at[pl.ds(base, per_w)], tgt_v, sem_t)
        cp.wait()
        ct.wait()

        zero = jnp.zeros((_LANES,), jnp.float32)

        def body(i, accs):
            b = i * (_UNROLL * _LANES)
            out = []
            for k in range(_UNROLL):
                p = pred_v[pl.ds(b + k * _LANES, _LANES)]
                tg = tgt_v[pl.ds(b + k * _LANES, _LANES)]
                out.append(accs[k] + _focal_vec(p, tg))
            return tuple(out)

        accs = lax.fori_loop(0, vecs, body, (zero,) * _UNROLL)
        acc_v[...] = (accs[0] + accs[1]) + (accs[2] + accs[3])
        pltpu.sync_copy(acc_v, out_hbm.at[wid])

    return sc_loss(predf, target)


def _tc_block(pred_ref, tgt_ref, out_ref):
    p = pred_ref[...]
    t = (tgt_ref[...] == 0).astype(jnp.float32)
    s = jax.nn.sigmoid(p)
    pt = (1.0 - s) * t + s * (1.0 - t)
    fw = (_ALPHA * t + (1.0 - _ALPHA) * (1.0 - t)) * pt * pt
    bce = jnp.maximum(p, 0.0) - p * t + jnp.log1p(jnp.exp(-jnp.abs(p)))
    part = jnp.sum(bce * fw)

    @pl.when(pl.program_id(0) == 0)
    def _():
        out_ref[0, 0] = 0.0

    out_ref[0, 0] += part


def _tc_partial_sum(predf, target, sc_n):
    n = predf.shape[0]
    blk = _TC_BLOCK_ROWS * _TC_COLS
    grid = (n - sc_n) // blk
    first = sc_n // blk
    return pl.pallas_call(
        _tc_block,
        grid=(grid,),
        in_specs=[
            pl.BlockSpec((blk,), lambda i: (first + i,)),
            pl.BlockSpec((blk,), lambda i: (first + i,)),
        ],
        out_specs=pl.BlockSpec(memory_space=pltpu.SMEM),
        out_shape=jax.ShapeDtypeStruct((1, 1), jnp.float32),
        compiler_params=pltpu.CompilerParams(
            dimension_semantics=("arbitrary",),
        ),
    )(predf, target)


def kernel(pred, target):
    n = pred.shape[0]
    predf = pred.reshape(n)
    blk = _TC_BLOCK_ROWS * _TC_COLS
    sc_n = (n * _SC_FRAC_NUM // _SC_FRAC_DEN) // blk * blk

    sc_partials = _sc_partial_sums(predf, target, sc_n)
    tc_partial = _tc_partial_sum(predf, target, sc_n)
    total = jnp.sum(sc_partials) + tc_partial[0, 0]
    return _LOSS_WEIGHT * (total / n)
